# Initial kernel scaffold; baseline (speedup 1.0000x reference)
import jax
import jax.numpy as jnp
from jax.experimental import pallas as pl


def kernel(userid, itemid, user_age, gender, user_occupation, item_kind, item_id_his, item_kind_his, label, user_table, item_table, age_table, gender_table, occ_table, kind_table, W1, b1, W2, b2, W3, b3, W4, b4, aW1, ab1, aW2, ab2, aW3, ab3):
    raise NotImplementedError("write your pallas kernel here")


# speedup vs baseline: 8.4884x; 8.4884x over previous
import functools

import jax
import jax.numpy as jnp
from jax import lax
from jax.experimental import pallas as pl
from jax.experimental.pallas import tpu as pltpu
from jax.experimental.pallas import tpu_sc as plsc

_B = 4096
_DIM = 16
_HIS = 50
_KIND = 10
_HID = (1 + _KIND) * _DIM
_R = 128
_GRID = _B // _R


_NW = 32
_G = 128
_GPG = 8
_GROUP = _G * _GPG


def _sc_stream(tbl, idx, out, nrows, wid, idx_v, rows_v, sem):
    ngroups = nrows // _GROUP
    trips = -(-ngroups // _NW)

    def group(g):
        gb = g * _GROUP
        pltpu.sync_copy(idx.at[pl.ds(gb, _GROUP)], idx_v)
        cps = [
            pltpu.async_copy(tbl.at[idx_v.at[pl.ds(j * _G, _G)]],
                             rows_v.at[pl.ds(j * _G, _G)], sem)
            for j in range(_GPG)
        ]
        for c in cps:
            c.wait()
        pltpu.sync_copy(rows_v, out.at[pl.ds(gb, _GROUP)])

    if ngroups <= _NW:
        @pl.when(wid < ngroups)
        def _():
            group(wid)
    else:
        def body(t, c):
            g = t * _NW + wid

            @pl.when(g < ngroups)
            def _():
                group(g)

            return c

        lax.fori_loop(0, trips, body, 0, unroll=False)


def _sc_gather_call(item_table, user_table, ct,
                    idx_hisK, idx_hisI, idx_item, idx_user, idx_kind,
                    idx_age, idx_gender, idx_occ):
    f32 = jnp.float32
    out_type = [
        jax.ShapeDtypeStruct((_B * _HIS * _KIND, 16), f32),
        jax.ShapeDtypeStruct((_B * _HIS, 16), f32),
        jax.ShapeDtypeStruct((_B, 16), f32),
        jax.ShapeDtypeStruct((_B, 16), f32),
        jax.ShapeDtypeStruct((_B * _KIND, 16), f32),
        jax.ShapeDtypeStruct((_B, 16), f32),
        jax.ShapeDtypeStruct((_B, 16), f32),
        jax.ShapeDtypeStruct((_B, 16), f32),
    ]
    mesh = plsc.VectorSubcoreMesh(core_axis_name="c", subcore_axis_name="s")

    @functools.partial(
        pl.kernel, mesh=mesh, out_type=out_type,
        compiler_params=pltpu.CompilerParams(use_tc_tiling_on_sc=False),
        scratch_types=[
            pltpu.VMEM((_GROUP,), jnp.int32),
            pltpu.VMEM((_GROUP, 16), f32),
            pltpu.SemaphoreType.DMA,
        ],
    )
    def sc_gather(item_t, user_t, ct_t,
                  i_hisK, i_hisI, i_item, i_user, i_kind, i_age, i_gen, i_occ,
                  o_hisK, o_hisI, o_item, o_user, o_kind, o_age, o_gen, o_occ,
                  idx_v, rows_v, sem):
        wid = lax.axis_index("s") * 2 + lax.axis_index("c")
        _sc_stream(ct_t, i_hisK, o_hisK, _B * _HIS * _KIND, wid,
                   idx_v, rows_v, sem)
        _sc_stream(item_t, i_hisI, o_hisI, _B * _HIS, wid, idx_v, rows_v, sem)
        _sc_stream(ct_t, i_kind, o_kind, _B * _KIND, wid, idx_v, rows_v, sem)
        _sc_stream(item_t, i_item, o_item, _B, wid, idx_v, rows_v, sem)
        _sc_stream(user_t, i_user, o_user, _B, wid, idx_v, rows_v, sem)
        _sc_stream(ct_t, i_age, o_age, _B, wid, idx_v, rows_v, sem)
        _sc_stream(ct_t, i_gen, o_gen, _B, wid, idx_v, rows_v, sem)
        _sc_stream(ct_t, i_occ, o_occ, _B, wid, idx_v, rows_v, sem)

    return sc_gather(item_table, user_table, ct,
                     idx_hisK, idx_hisI, idx_item, idx_user, idx_kind,
                     idx_age, idx_gender, idx_occ)


def _tc_body(hisK_ref, hisI_ref, itemw_ref, kindw_ref, userw_ref, agew_ref,
             genderw_ref, occw_ref, label_ref, P_ref, Q_ref, ab1_ref, aW2_ref,
             ab2_ref, aW3_ref, ab3_ref, W1_ref, b1_ref, W2_ref, b2_ref,
             W3_ref, b3_ref, W4_ref, b4_ref, out_ref):
    i = pl.program_id(0)
    f32 = jnp.float32
    hisK = hisK_ref[...]
    hisI = hisI_ref[...]
    itemw = itemw_ref[...]
    kindw = kindw_ref[...]
    P = P_ref[...]
    Q = Q_ref[...]

    dot = lambda a, b: jnp.dot(a, b, preferred_element_type=f32)
    u = dot(hisI, P[0:16, :]) + dot(hisK, P[16:176, :])
    v = dot(itemw, Q[0:16, :]) + dot(kindw, Q[16:176, :])
    h1 = jax.nn.relu(u.reshape(_R, _HIS, 64) + v[:, None, :] + ab1_ref[...])
    h1f = h1.reshape(_R * _HIS, 64)
    h2 = jax.nn.relu(dot(h1f, aW2_ref[...]) + ab2_ref[...])
    io = dot(h2, aW3_ref[...]) + ab3_ref[...]

    wI = hisI * hisI * io
    wK = hisK * hisK * io
    poolI = wI.reshape(_R, _HIS, 16).sum(axis=1)
    poolK = wK.reshape(_R, _HIS, 160).sum(axis=1)

    W1 = W1_ref[...]
    af = (dot(userw_ref[...], W1[0:16, :]) + dot(itemw, W1[16:32, :])
          + dot(agew_ref[...], W1[32:48, :]) + dot(genderw_ref[...], W1[48:64, :])
          + dot(occw_ref[...], W1[64:80, :]) + dot(kindw, W1[80:240, :])
          + dot(poolI, W1[240:256, :]) + dot(poolK, W1[256:416, :])
          + b1_ref[...])
    h = jax.nn.relu(af)
    h = jax.nn.relu(dot(h, W2_ref[...]) + b2_ref[...])
    h = jax.nn.relu(dot(h, W3_ref[...]) + b3_ref[...])
    logit = jax.nn.sigmoid(dot(h, W4_ref[...]) + b4_ref[...])
    lab = label_ref[...]
    ll = -(lab * jnp.log(logit + 1e-6) + (1.0 - lab) * jnp.log(1.0 - logit + 1e-6))
    part = (ll.sum() * (1.0 / _B)).reshape(1, 1)
    acc = jnp.where(i == 0, jnp.zeros((1, 1), f32), out_ref[...])
    out_ref[...] = acc + part


def _tc_specs():
    bcast = lambda shape: pl.BlockSpec(shape, lambda i: (0, 0))
    row = lambda shape: pl.BlockSpec(shape, lambda i: (i, 0))
    in_specs = [
        row((_R * _HIS, 160)),
        row((_R * _HIS, 16)),
        row((_R, 16)),
        row((_R, 160)),
        row((_R, 16)),
        row((_R, 16)),
        row((_R, 16)),
        row((_R, 16)),
        row((_R, 1)),
        bcast((_HID, 64)),
        bcast((_HID, 64)),
        bcast((1, 64)),
        bcast((64, 32)),
        bcast((1, 32)),
        bcast((32, 1)),
        bcast((1, 1)),
        bcast((416, 128)),
        bcast((1, 128)),
        bcast((128, 64)),
        bcast((1, 64)),
        bcast((64, 32)),
        bcast((1, 32)),
        bcast((32, 1)),
        bcast((1, 1)),
    ]
    out_spec = pl.BlockSpec((1, 1), lambda i: (0, 0))
    return in_specs, out_spec


def _tc_call(*args):
    in_specs, out_spec = _tc_specs()
    return pl.pallas_call(
        _tc_body,
        grid=(_GRID,),
        in_specs=in_specs,
        out_specs=out_spec,
        out_shape=jax.ShapeDtypeStruct((1, 1), jnp.float32),
    )(*args)


def kernel(userid, itemid, user_age, gender, user_occupation, item_kind,
           item_id_his, item_kind_his, label, user_table, item_table,
           age_table, gender_table, occ_table, kind_table, W1, b1, W2, b2,
           W3, b3, W4, b4, aW1, ab1, aW2, ab2, aW3, ab3):
    kind_z = kind_table.at[0].set(0.0)
    ct = jnp.concatenate([age_table, gender_table, occ_table, kind_z], axis=0)
    i32 = jnp.int32

    (hisK16, hisI, itemw, userw, kindw16, agew, genderw, occw) = _sc_gather_call(
        item_table, user_table, ct,
        (item_kind_his.astype(i32) + 3000).reshape(-1),
        item_id_his.astype(i32).reshape(-1),
        itemid.astype(i32).reshape(-1),
        userid.astype(i32).reshape(-1),
        (item_kind.astype(i32) + 3000).reshape(-1),
        user_age.astype(i32).reshape(-1),
        (gender.astype(i32) + 1000).reshape(-1),
        (user_occupation.astype(i32) + 2000).reshape(-1),
    )
    hisK = hisK16.reshape(_B * _HIS, 160)
    kindw = kindw16.reshape(_B, 160)

    A1 = aW1[0:_HID]
    A2 = aW1[_HID:2 * _HID]
    A3 = aW1[2 * _HID:3 * _HID]
    P = A1 + A2
    Q = A3 - A2

    loss = _tc_call(
        hisK, hisI, itemw, kindw, userw, agew, genderw, occw, label,
        P, Q, ab1.reshape(1, -1), aW2, ab2.reshape(1, -1), aW3,
        ab3.reshape(1, -1), W1, b1.reshape(1, -1), W2, b2.reshape(1, -1),
        W3, b3.reshape(1, -1), W4, b4.reshape(1, -1))
    return loss[0, 0]


# speedup vs baseline: 8.9658x; 1.0562x over previous
import functools

import jax
import jax.numpy as jnp
from jax import lax
from jax.experimental import pallas as pl
from jax.experimental.pallas import tpu as pltpu
from jax.experimental.pallas import tpu_sc as plsc

_B = 4096
_DIM = 16
_HIS = 50
_KIND = 10
_HID = (1 + _KIND) * _DIM
_R = 128
_GRID = _B // _R


_NW = 32
_G = 128
_GPG = 16
_GROUP = _G * _GPG


def _sc_stream(tbl, idx, out, nrows, wid, idx_v, rows_v, sem_g, sem_w0,
               sem_w1):
    ngroups = nrows // _GROUP
    trips = -(-ngroups // _NW)
    sems = (sem_w0, sem_w1)

    def fire(g, half, sem_w):
        gb = g * _GROUP
        base = half * _GROUP
        pltpu.sync_copy(idx.at[pl.ds(gb, _GROUP)], idx_v)
        cps = [
            pltpu.async_copy(tbl.at[idx_v.at[pl.ds(j * _G, _G)]],
                             rows_v.at[pl.ds(base + j * _G, _G)], sem_g)
            for j in range(_GPG)
        ]
        for c in cps:
            c.wait()
        pltpu.async_copy(rows_v.at[pl.ds(base, _GROUP)],
                         out.at[pl.ds(gb, _GROUP)], sem_w)

    def drain(half, sem_w):
        pltpu.make_async_copy(out.at[pl.ds(0, _GROUP)],
                              rows_v.at[pl.ds(half * _GROUP, _GROUP)],
                              sem_w).wait()

    def pair_body(u, c):
        for half in (0, 1):
            t = 2 * u + half
            g = t * _NW + wid
            gprev = (t - 2) * _NW + wid

            drain_ok = jnp.logical_and(
                jnp.logical_and(t >= 2, t < trips), gprev < ngroups)

            @pl.when(drain_ok)
            def _():
                drain(half, sems[half])

            @pl.when(g < ngroups)
            def _():
                fire(g, half, sems[half])

        return c

    lax.fori_loop(0, (trips + 1) // 2, pair_body, 0, unroll=False)
    for tt in (trips - 2, trips - 1):
        if tt >= 0:
            g = tt * _NW + wid

            @pl.when(g < ngroups)
            def _():
                drain(tt % 2, sems[tt % 2])


def _sc_gather_call(item_table, user_table, ct,
                    idx_hisK, idx_hisI, idx_item, idx_user, idx_kind,
                    idx_age, idx_gender, idx_occ):
    f32 = jnp.float32
    out_type = [
        jax.ShapeDtypeStruct((_B * _HIS * _KIND, 16), f32),
        jax.ShapeDtypeStruct((_B * _HIS, 16), f32),
        jax.ShapeDtypeStruct((_B, 16), f32),
        jax.ShapeDtypeStruct((_B, 16), f32),
        jax.ShapeDtypeStruct((_B * _KIND, 16), f32),
        jax.ShapeDtypeStruct((_B, 16), f32),
        jax.ShapeDtypeStruct((_B, 16), f32),
        jax.ShapeDtypeStruct((_B, 16), f32),
    ]
    mesh = plsc.VectorSubcoreMesh(core_axis_name="c", subcore_axis_name="s")

    @functools.partial(
        pl.kernel, mesh=mesh, out_type=out_type,
        compiler_params=pltpu.CompilerParams(use_tc_tiling_on_sc=False),
        scratch_types=[
            pltpu.VMEM((_GROUP,), jnp.int32),
            pltpu.VMEM((2 * _GROUP, 16), f32),
            pltpu.SemaphoreType.DMA,
            pltpu.SemaphoreType.DMA,
            pltpu.SemaphoreType.DMA,
        ],
    )
    def sc_gather(item_t, user_t, ct_t,
                  i_hisK, i_hisI, i_item, i_user, i_kind, i_age, i_gen, i_occ,
                  o_hisK, o_hisI, o_item, o_user, o_kind, o_age, o_gen, o_occ,
                  idx_v, rows_v, sg, sw0, sw1):
        wid = lax.axis_index("s") * 2 + lax.axis_index("c")
        args = (wid, idx_v, rows_v, sg, sw0, sw1)
        _sc_stream(ct_t, i_hisK, o_hisK, _B * _HIS * _KIND, *args)
        _sc_stream(item_t, i_hisI, o_hisI, _B * _HIS, *args)
        _sc_stream(ct_t, i_kind, o_kind, _B * _KIND, *args)
        _sc_stream(item_t, i_item, o_item, _B, *args)
        _sc_stream(user_t, i_user, o_user, _B, *args)
        _sc_stream(ct_t, i_age, o_age, _B, *args)
        _sc_stream(ct_t, i_gen, o_gen, _B, *args)
        _sc_stream(ct_t, i_occ, o_occ, _B, *args)

    return sc_gather(item_table, user_table, ct,
                     idx_hisK, idx_hisI, idx_item, idx_user, idx_kind,
                     idx_age, idx_gender, idx_occ)


def _tc_body(hisK_ref, hisI_ref, itemw_ref, kindw_ref, userw_ref, agew_ref,
             genderw_ref, occw_ref, label_ref, S_ref, St_ref, P_ref, Q_ref,
             ab1_ref, aW2_ref, ab2_ref, aW3_ref, ab3_ref, W1_ref, b1_ref,
             W2_ref, b2_ref, W3_ref, b3_ref, W4_ref, b4_ref, out_ref):
    i = pl.program_id(0)
    f32 = jnp.float32
    hisK = hisK_ref[...]
    hisI = hisI_ref[...]
    itemw = itemw_ref[...]
    kindw = kindw_ref[...]
    P = P_ref[...]
    Q = Q_ref[...]
    S = S_ref[...]
    St = St_ref[...]

    dot = lambda a, b: jnp.dot(a, b, preferred_element_type=f32)
    u = dot(hisI, P[0:16, :]) + dot(hisK, P[16:176, :])
    v = dot(itemw, Q[0:16, :]) + dot(kindw, Q[16:176, :]) + ab1_ref[...]
    h1 = jax.nn.relu(u + dot(S, v))
    h2 = jax.nn.relu(dot(h1, aW2_ref[...]) + ab2_ref[...])
    io = dot(h2, aW3_ref[...]) + ab3_ref[...]

    wI = hisI * hisI * io
    wK = hisK * hisK * io
    poolI = dot(St, wI)
    poolK = dot(St, wK)

    W1 = W1_ref[...]
    af = (dot(userw_ref[...], W1[0:16, :]) + dot(itemw, W1[16:32, :])
          + dot(agew_ref[...], W1[32:48, :]) + dot(genderw_ref[...], W1[48:64, :])
          + dot(occw_ref[...], W1[64:80, :]) + dot(kindw, W1[80:240, :])
          + dot(poolI, W1[240:256, :]) + dot(poolK, W1[256:416, :])
          + b1_ref[...])
    h = jax.nn.relu(af)
    h = jax.nn.relu(dot(h, W2_ref[...]) + b2_ref[...])
    h = jax.nn.relu(dot(h, W3_ref[...]) + b3_ref[...])
    logit = jax.nn.sigmoid(dot(h, W4_ref[...]) + b4_ref[...])
    lab = label_ref[...]
    ll = -(lab * jnp.log(logit + 1e-6) + (1.0 - lab) * jnp.log(1.0 - logit + 1e-6))
    part = (ll.sum() * (1.0 / _B)).reshape(1, 1)
    acc = jnp.where(i == 0, jnp.zeros((1, 1), f32), out_ref[...])
    out_ref[...] = acc + part


def _tc_specs():
    bcast = lambda shape: pl.BlockSpec(shape, lambda i: (0, 0))
    row = lambda shape: pl.BlockSpec(shape, lambda i: (i, 0))
    in_specs = [
        row((_R * _HIS, 160)),
        row((_R * _HIS, 16)),
        row((_R, 16)),
        row((_R, 160)),
        row((_R, 16)),
        row((_R, 16)),
        row((_R, 16)),
        row((_R, 16)),
        row((_R, 1)),
        bcast((_R * _HIS, _R)),
        bcast((_R, _R * _HIS)),
        bcast((_HID, 64)),
        bcast((_HID, 64)),
        bcast((1, 64)),
        bcast((64, 32)),
        bcast((1, 32)),
        bcast((32, 1)),
        bcast((1, 1)),
        bcast((416, 128)),
        bcast((1, 128)),
        bcast((128, 64)),
        bcast((1, 64)),
        bcast((64, 32)),
        bcast((1, 32)),
        bcast((32, 1)),
        bcast((1, 1)),
    ]
    out_spec = pl.BlockSpec((1, 1), lambda i: (0, 0))
    return in_specs, out_spec


def _tc_call(*args):
    in_specs, out_spec = _tc_specs()
    return pl.pallas_call(
        _tc_body,
        grid=(_GRID,),
        in_specs=in_specs,
        out_specs=out_spec,
        out_shape=jax.ShapeDtypeStruct((1, 1), jnp.float32),
    )(*args)


def kernel(userid, itemid, user_age, gender, user_occupation, item_kind,
           item_id_his, item_kind_his, label, user_table, item_table,
           age_table, gender_table, occ_table, kind_table, W1, b1, W2, b2,
           W3, b3, W4, b4, aW1, ab1, aW2, ab2, aW3, ab3):
    kind_z = kind_table.at[0].set(0.0)
    ct = jnp.concatenate([age_table, gender_table, occ_table, kind_z], axis=0)
    i32 = jnp.int32

    (hisK16, hisI, itemw, userw, kindw16, agew, genderw, occw) = _sc_gather_call(
        item_table, user_table, ct,
        (item_kind_his.astype(i32) + 3000).reshape(-1),
        item_id_his.astype(i32).reshape(-1),
        itemid.astype(i32).reshape(-1),
        userid.astype(i32).reshape(-1),
        (item_kind.astype(i32) + 3000).reshape(-1),
        user_age.astype(i32).reshape(-1),
        (gender.astype(i32) + 1000).reshape(-1),
        (user_occupation.astype(i32) + 2000).reshape(-1),
    )
    hisK = hisK16.reshape(_B * _HIS, 160)
    kindw = kindw16.reshape(_B, 160)

    A1 = aW1[0:_HID]
    A2 = aW1[_HID:2 * _HID]
    A3 = aW1[2 * _HID:3 * _HID]
    P = A1 + A2
    Q = A3 - A2
    S = jnp.repeat(jnp.eye(_R, dtype=jnp.float32), _HIS, axis=0)
    St = S.T

    loss = _tc_call(
        hisK, hisI, itemw, kindw, userw, agew, genderw, occw, label,
        S, St, P, Q, ab1.reshape(1, -1), aW2, ab2.reshape(1, -1), aW3,
        ab3.reshape(1, -1), W1, b1.reshape(1, -1), W2, b2.reshape(1, -1),
        W3, b3.reshape(1, -1), W4, b4.reshape(1, -1))
    return loss[0, 0]
